# Initial kernel scaffold; baseline (speedup 1.0000x reference)
import jax
import jax.numpy as jnp
from jax.experimental import pallas as pl


def kernel(logits, temperatures, top_ks, top_ps, min_ps, u):
    raise NotImplementedError("write your pallas kernel here")


# speedup vs baseline: 6.3831x; 6.3831x over previous
import functools

import jax
import jax.numpy as jnp
from jax import lax
from jax.experimental import pallas as pl
from jax.experimental.pallas import tpu as pltpu
from jax.experimental.pallas import tpu_sc as plsc

B = 128
V = 100000
NB = 1024
CH = 10000
CBUF = 4352
CAND = 4096
SLACK = 64
ROWS_PER_W = 4
CHUNK_A = 7696
NCH_A = (V + CHUNK_A - 1) // CHUNK_A


def _phase_a_body(l_ref, t_ref, m_ref, z_ref, e_scr):
    ph = pl.program_id(0)
    ci = pl.program_id(1)
    row = lax.broadcasted_iota(jnp.int32, (CHUNK_A, B), 0) + ci * CHUNK_A
    valid = row < V
    t = t_ref[...][0:1, :]
    s = l_ref[...] / t

    @pl.when(jnp.logical_and(ph == 0, ci == 0))
    def _():
        m_ref[...] = jnp.full((8, B), -jnp.inf, jnp.float32)

    @pl.when(ph == 0)
    def _():
        sm = jnp.where(valid, s, -jnp.inf)
        cmax = jnp.max(sm, axis=0, keepdims=True)
        m_ref[...] = jnp.maximum(m_ref[...], jnp.broadcast_to(cmax, (8, B)))

    @pl.when(jnp.logical_and(ph == 1, ci == 0))
    def _():
        z_ref[...] = jnp.zeros((8, B), jnp.float32)

    @pl.when(ph == 1)
    def _():
        m = m_ref[...][0:1, :]
        e_scr[...] = jnp.where(valid, jnp.exp(s - m), 0.0)

        def body(j, acc):
            return acc + e_scr[pl.ds(j * 8, 8), :]
        acc = lax.fori_loop(0, CHUNK_A // 8, body,
                            jnp.zeros((8, B), jnp.float32))
        bb = acc + pltpu.roll(acc, 4, 0)
        cc = bb + pltpu.roll(bb, 2, 0)
        dd = cc + pltpu.roll(cc, 1, 0)
        z_ref[...] = z_ref[...] + jnp.broadcast_to(dd[0:1, :], (8, B))


def _phase_a(logits_t, t8):
    return pl.pallas_call(
        _phase_a_body,
        grid=(2, NCH_A),
        in_specs=[
            pl.BlockSpec((CHUNK_A, B), lambda p, i: (i, 0)),
            pl.BlockSpec((8, B), lambda p, i: (0, 0)),
        ],
        out_specs=[
            pl.BlockSpec((8, B), lambda p, i: (0, 0)),
            pl.BlockSpec((8, B), lambda p, i: (0, 0)),
        ],
        out_shape=[
            jax.ShapeDtypeStruct((8, B), jnp.float32),
            jax.ShapeDtypeStruct((8, B), jnp.float32),
        ],
        scratch_shapes=[pltpu.VMEM((CHUNK_A, B), jnp.float32)],
    )(logits_t, t8)


def _sc_select_body(lg, m16, t16, k16, cl, ci, cnt,
                    chunk, hist, clb, cib, msc, tsc, ksc, csc):
    wid = lax.axis_index("s") * 2 + lax.axis_index("c")

    def row_body(r, _carry):
        row = wid * ROWS_PER_W + r
        base = row * V
        pltpu.sync_copy(m16.at[row], msc)
        pltpu.sync_copy(t16.at[row], tsc)
        pltpu.sync_copy(k16.at[row], ksc)
        m_vec = msc[...]
        t_vec = tsc[...]
        k_vec = ksc[...]

        def bucket_of(lv):
            v = m_vec - lv / t_vec
            bk = (v * jnp.float32(NB / 48.0)).astype(jnp.int32)
            return jnp.clip(bk, 0, NB - 1)

        def zb(i, c):
            hist[pl.ds(i * 16, 16)] = jnp.zeros((16,), jnp.int32)
            return c
        lax.fori_loop(0, NB // 16, zb, 0)

        ones = jnp.ones((16,), jnp.int32)

        def chunk_hist(c, carry):
            pltpu.sync_copy(lg.at[pl.ds(base + c * CH, CH)], chunk)

            def vb(i, cc):
                bk = bucket_of(chunk[pl.ds(i * 16, 16)])
                plsc.addupdate_scatter(hist, [bk], ones)
                return cc
            return lax.fori_loop(0, CH // 16, vb, carry)
        lax.fori_loop(0, V // CH, chunk_hist, 0)

        sentinel = jnp.full((16,), NB, jnp.int32)

        def scan_body(i, carry):
            csum, bstar = carry
            h = hist[pl.ds(i * 16, 16)]
            hc = plsc.cumsum(h) + csum
            cond = hc >= k_vec
            anyv = plsc.all_reduce_population_count(cond) > 0
            ffs = plsc.all_reduce_ffs(cond)
            cand_b = ffs + i * 16
            bstar = jnp.where(jnp.logical_and(bstar == NB, anyv), cand_b, bstar)
            return csum + jnp.sum(h), bstar
        _, bstar = lax.fori_loop(0, NB // 16, scan_body,
                                 (jnp.int32(0), sentinel))

        def chunk_sel(c, off):
            pltpu.sync_copy(lg.at[pl.ds(base + c * CH, CH)], chunk)

            def vb(i, off):
                lv = chunk[pl.ds(i * 16, 16)]
                mask = bucket_of(lv) <= bstar
                idxv = lax.iota(jnp.int32, 16) + (c * CH + i * 16)
                offc = jnp.minimum(off, CBUF - 16)
                plsc.store_compressed(clb.at[pl.ds(offc, 16)], lv, mask=mask)
                plsc.store_compressed(cib.at[pl.ds(offc, 16)], idxv, mask=mask)
                return off + jnp.sum(mask.astype(jnp.int32))
            return lax.fori_loop(0, CH // 16, vb, off)
        off = lax.fori_loop(0, V // CH, chunk_sel, jnp.int32(0))

        pltpu.sync_copy(clb, cl.at[pl.ds(row * CBUF, CBUF)])
        pltpu.sync_copy(cib, ci.at[pl.ds(row * CBUF, CBUF)])
        csc[...] = jnp.zeros((16,), jnp.int32) + off
        pltpu.sync_copy(csc, cnt.at[row])
        return _carry

    lax.fori_loop(0, ROWS_PER_W, row_body, 0)


def _sc_select(lg_flat, m16, t16, k16):
    mesh = plsc.VectorSubcoreMesh(core_axis_name="c", subcore_axis_name="s")
    fn = functools.partial(
        pl.kernel,
        mesh=mesh,
        compiler_params=pltpu.CompilerParams(needs_layout_passes=False),
        out_type=[
            jax.ShapeDtypeStruct((B * CBUF,), jnp.float32),
            jax.ShapeDtypeStruct((B * CBUF,), jnp.int32),
            jax.ShapeDtypeStruct((B, 16), jnp.int32),
        ],
        scratch_types=[
            pltpu.VMEM((CH,), jnp.float32),
            pltpu.VMEM((NB,), jnp.int32),
            pltpu.VMEM((CBUF,), jnp.float32),
            pltpu.VMEM((CBUF,), jnp.int32),
            pltpu.VMEM((16,), jnp.float32),
            pltpu.VMEM((16,), jnp.float32),
            pltpu.VMEM((16,), jnp.int32),
            pltpu.VMEM((16,), jnp.int32),
        ],
    )(_sc_select_body)
    return fn(lg_flat, m16, t16, k16)


RB = 16
def _ref_cumsum(x):
    nblk = CAND // 128
    R = RB * nblk
    xb = jnp.reshape(x, (R, 128))
    lane = lax.broadcasted_iota(jnp.int32, (R, 128), 1)
    bm = lax.broadcasted_iota(jnp.int32, (R, 128), 0) & (nblk - 1)
    acc = jnp.broadcast_to(xb[:, 0:1], (R, 128))
    for t in range(1, 128):
        bv = jnp.broadcast_to(xb[:, t:t + 1], (R, 128))
        acc = acc + jnp.where(lane >= t, bv, 0.0)
    tot = jnp.broadcast_to(acc[:, 127:128], (R, 128))
    off = jnp.zeros((R, 128), jnp.float32)
    for t in range(nblk - 1):
        w = jnp.where(bm == t, tot, 0.0)
        for sp in (1, 2, 4, 8, 16):
            w = w + pltpu.roll(w, sp, 0)
        off = off + jnp.where(bm > t, w, 0.0)
    return jnp.reshape(acc + off, (RB, CAND))


def _phase_c_body(cl_ref, ci_ref, cnt_ref, m_ref, z_ref, t_ref,
                  tk_ref, tp_ref, mp_ref, u_ref, o_ref):
    lane = lax.broadcasted_iota(jnp.int32, (RB, CAND), 1)
    cnt = cnt_ref[...][:, 0:1]
    t = t_ref[...][:, 0:1]
    m = m_ref[...][:, 0:1]
    valid = lane < cnt

    s = cl_ref[...] / t
    e = jnp.where(valid, jnp.exp(s - m), 0.0)
    z = z_ref[...][:, 0:1]
    p = jnp.where(valid, e / z, 0.0)
    ix = jnp.where(valid, ci_ref[...], jnp.int32(0x7FFFFFFF))

    for k in [2 << i for i in range(12)]:
        d = k // 2
        while d >= 1:
            pu = pltpu.roll(p, CAND - d, 1)
            pdn = pltpu.roll(p, d, 1)
            iu = pltpu.roll(ix, CAND - d, 1)
            idn = pltpu.roll(ix, d, 1)
            up_sel = (lane & d) == 0
            pb = jnp.where(up_sel, pu, pdn)
            ib = jnp.where(up_sel, iu, idn)
            dir_i = ((lane & k) == 0) == up_sel
            self_wins = jnp.logical_or(
                p > pb, jnp.logical_and(p == pb, ix < ib))
            keep = dir_i == self_wins
            p = jnp.where(keep, p, pb)
            ix = jnp.where(keep, ix, ib)
            d //= 2

    psum = _ref_cumsum(p)
    pmax = p[:, 0:1]
    p1 = jnp.where((psum - p) > tp_ref[...][:, 0:1], 0.0, p)
    p2 = jnp.where(lane >= tk_ref[...][:, 0:1], 0.0, p1)
    p3 = jnp.where(p2 < pmax * mp_ref[...][:, 0:1], 0.0, p2)
    mx = jnp.max(p3, axis=1, keepdims=True)
    pr = p3 / mx
    cdf = _ref_cumsum(pr)
    target = u_ref[...][:, 0:1] * cdf[:, CAND - 1:CAND]
    j = jnp.sum((cdf < target).astype(jnp.int32), axis=1, keepdims=True)
    j = jnp.clip(j, 0, V - 1)
    tok = jnp.sum(jnp.where(lane == j, ix, 0), axis=1, keepdims=True)
    o_ref[...] = jnp.broadcast_to(tok, (RB, 128))


def _phase_c(cl, ci, cnt, m128, z128, t128, tk128, tp128, mp128, u128):
    spec_cand = pl.BlockSpec((RB, CAND), lambda i: (i, 0))
    spec_128 = pl.BlockSpec((RB, 128), lambda i: (i, 0))
    spec_16 = pl.BlockSpec((RB, 16), lambda i: (i, 0))
    return pl.pallas_call(
        _phase_c_body,
        grid=(B // RB,),
        in_specs=[spec_cand, spec_cand, spec_16, spec_128, spec_128,
                  spec_128, spec_128, spec_128, spec_128, spec_128],
        out_specs=spec_128,
        out_shape=jax.ShapeDtypeStruct((B, 128), jnp.int32),
    )(cl, ci, cnt, m128, z128, t128, tk128, tp128, mp128, u128)


def kernel(logits, temperatures, top_ks, top_ps, min_ps, u):
    t128 = jnp.broadcast_to(temperatures.astype(jnp.float32), (B, 128))
    t8 = jnp.broadcast_to(jnp.reshape(temperatures.astype(jnp.float32),
                                      (1, B)), (8, B))
    m8, z8 = _phase_a(logits.T, t8)
    m128 = jnp.broadcast_to(m8[0][:, None], (B, 128))
    z128 = jnp.broadcast_to(z8[0][:, None], (B, 128))

    m16 = m128[:, :16]
    t16 = t128[:, :16]
    k16 = jnp.broadcast_to(
        (top_ks.astype(jnp.int32) + SLACK)[:, None], (B, 16))
    cl, ci, cnt = _sc_select(logits.reshape(-1), m16, t16, k16)
    cl = cl.reshape(B, CBUF)[:, :CAND]
    ci = ci.reshape(B, CBUF)[:, :CAND]

    tk128 = jnp.broadcast_to(top_ks.astype(jnp.int32)[:, None], (B, 128))
    tp128 = jnp.broadcast_to(top_ps.astype(jnp.float32)[:, None], (B, 128))
    mp128 = jnp.broadcast_to(min_ps.astype(jnp.float32)[:, None], (B, 128))
    u128 = jnp.broadcast_to(u.astype(jnp.float32)[:, None], (B, 128))

    out = _phase_c(cl, ci, cnt, m128, z128, t128, tk128, tp128, mp128, u128)
    return out[:, 0]


# speedup vs baseline: 7.9259x; 1.2417x over previous
import functools

import jax
import jax.numpy as jnp
from jax import lax
from jax.experimental import pallas as pl
from jax.experimental.pallas import tpu as pltpu
from jax.experimental.pallas import tpu_sc as plsc

B = 128
V = 100000
NB = 1024
CH = 10000
CBUF = 4352
CAND = 4096
SLACK = 64
ROWS_PER_W = 4
CHUNK_A = 7696
NCH_A = (V + CHUNK_A - 1) // CHUNK_A


def _phase_a_body(l_ref, t_ref, m_ref, z_ref, e_scr):
    ph = pl.program_id(0)
    ci = pl.program_id(1)
    row = lax.broadcasted_iota(jnp.int32, (CHUNK_A, B), 0) + ci * CHUNK_A
    valid = row < V
    t = t_ref[...][0:1, :]
    s = l_ref[...] / t

    @pl.when(jnp.logical_and(ph == 0, ci == 0))
    def _():
        m_ref[...] = jnp.full((8, B), -jnp.inf, jnp.float32)

    @pl.when(ph == 0)
    def _():
        sm = jnp.where(valid, s, -jnp.inf)
        cmax = jnp.max(sm, axis=0, keepdims=True)
        m_ref[...] = jnp.maximum(m_ref[...], jnp.broadcast_to(cmax, (8, B)))

    @pl.when(jnp.logical_and(ph == 1, ci == 0))
    def _():
        z_ref[...] = jnp.zeros((8, B), jnp.float32)

    @pl.when(ph == 1)
    def _():
        m = m_ref[...][0:1, :]
        e_scr[...] = jnp.where(valid, jnp.exp(s - m), 0.0)

        def body(j, acc):
            return acc + e_scr[pl.ds(j * 8, 8), :]
        acc = lax.fori_loop(0, CHUNK_A // 8, body,
                            jnp.zeros((8, B), jnp.float32))
        bb = acc + pltpu.roll(acc, 4, 0)
        cc = bb + pltpu.roll(bb, 2, 0)
        dd = cc + pltpu.roll(cc, 1, 0)
        z_ref[...] = z_ref[...] + jnp.broadcast_to(dd[0:1, :], (8, B))


def _phase_a(logits_t, t8):
    return pl.pallas_call(
        _phase_a_body,
        grid=(2, NCH_A),
        in_specs=[
            pl.BlockSpec((CHUNK_A, B), lambda p, i: (i, 0)),
            pl.BlockSpec((8, B), lambda p, i: (0, 0)),
        ],
        out_specs=[
            pl.BlockSpec((8, B), lambda p, i: (0, 0)),
            pl.BlockSpec((8, B), lambda p, i: (0, 0)),
        ],
        out_shape=[
            jax.ShapeDtypeStruct((8, B), jnp.float32),
            jax.ShapeDtypeStruct((8, B), jnp.float32),
        ],
        scratch_shapes=[pltpu.VMEM((CHUNK_A, B), jnp.float32)],
    )(logits_t, t8)


def _sc_select_body(lg, mt16, k16, cl, ci, cnt,
                    rowbuf, hist, clb, cib, msc, ksc, csc):
    wid = lax.axis_index("s") * 2 + lax.axis_index("c")
    scale = jnp.float32(NB / 72.0)

    def row_body(r, _carry):
        row = wid * ROWS_PER_W + r
        base = row * V
        pltpu.sync_copy(mt16.at[row], msc)
        pltpu.sync_copy(k16.at[row], ksc)
        mt_vec = msc[...]
        k_vec = ksc[...]
        pltpu.sync_copy(lg.at[pl.ds(base, V)], rowbuf)

        def bucket_of(lv):
            bk = ((mt_vec - lv) * scale).astype(jnp.int32)
            return jnp.clip(bk, 0, NB - 1)

        def zb(i, c):
            hist[pl.ds(i * 16, 16)] = jnp.zeros((16,), jnp.int32)
            return c
        lax.fori_loop(0, NB // 16, zb, 0)

        ones = jnp.ones((16,), jnp.int32)

        def vb_hist(i, cc):
            bk = bucket_of(rowbuf[pl.ds(i * 16, 16)])
            plsc.addupdate_scatter(hist, [bk], ones)
            return cc
        lax.fori_loop(0, V // 16, vb_hist, 0)

        sentinel = jnp.full((16,), NB, jnp.int32)

        def scan_body(i, carry):
            csum, bstar = carry
            h = hist[pl.ds(i * 16, 16)]
            hc = plsc.cumsum(h) + csum
            cond = hc >= k_vec
            anyv = plsc.all_reduce_population_count(cond) > 0
            ffs = plsc.all_reduce_ffs(cond)
            cand_b = ffs + i * 16
            bstar = jnp.where(jnp.logical_and(bstar == NB, anyv), cand_b, bstar)
            return csum + jnp.sum(h), bstar
        _, bstar = lax.fori_loop(0, NB // 16, scan_body,
                                 (jnp.int32(0), sentinel))

        def vb_sel(i, off):
            lv = rowbuf[pl.ds(i * 16, 16)]
            mask = bucket_of(lv) <= bstar
            idxv = lax.iota(jnp.int32, 16) + i * 16
            offc = jnp.minimum(off, CBUF - 16)
            plsc.store_compressed(clb.at[pl.ds(offc, 16)], lv, mask=mask)
            plsc.store_compressed(cib.at[pl.ds(offc, 16)], idxv, mask=mask)
            return off + jnp.sum(mask.astype(jnp.int32))
        off = lax.fori_loop(0, V // 16, vb_sel, jnp.int32(0))

        pltpu.sync_copy(clb, cl.at[pl.ds(row * CBUF, CBUF)])
        pltpu.sync_copy(cib, ci.at[pl.ds(row * CBUF, CBUF)])
        csc[...] = jnp.zeros((16,), jnp.int32) + off
        pltpu.sync_copy(csc, cnt.at[row])
        return _carry

    lax.fori_loop(0, ROWS_PER_W, row_body, 0)


def _sc_select(lg_flat, mt16, k16):
    mesh = plsc.VectorSubcoreMesh(core_axis_name="c", subcore_axis_name="s")
    fn = functools.partial(
        pl.kernel,
        mesh=mesh,
        compiler_params=pltpu.CompilerParams(needs_layout_passes=False),
        out_type=[
            jax.ShapeDtypeStruct((B * CBUF,), jnp.float32),
            jax.ShapeDtypeStruct((B * CBUF,), jnp.int32),
            jax.ShapeDtypeStruct((B, 16), jnp.int32),
        ],
        scratch_types=[
            pltpu.VMEM((V,), jnp.float32),
            pltpu.VMEM((NB,), jnp.int32),
            pltpu.VMEM((CBUF,), jnp.float32),
            pltpu.VMEM((CBUF,), jnp.int32),
            pltpu.VMEM((16,), jnp.float32),
            pltpu.VMEM((16,), jnp.int32),
            pltpu.VMEM((16,), jnp.int32),
        ],
    )(_sc_select_body)
    return fn(lg_flat, mt16, k16)


RB = 32
HALF = CAND // 2
def _ref_cumsum(x, ncols):
    nblk = ncols // 128
    R = RB * nblk
    xb = jnp.reshape(x, (R, 128))
    lane = lax.broadcasted_iota(jnp.int32, (R, 128), 1)
    bm = lax.broadcasted_iota(jnp.int32, (R, 128), 0) & (nblk - 1)
    acc = jnp.broadcast_to(xb[:, 0:1], (R, 128))
    for t in range(1, 128):
        bv = jnp.broadcast_to(xb[:, t:t + 1], (R, 128))
        acc = acc + jnp.where(lane >= t, bv, 0.0)
    tot = jnp.broadcast_to(acc[:, 127:128], (R, 128))
    off = jnp.zeros((R, 128), jnp.float32)
    for t in range(nblk - 1):
        w = jnp.where(bm == t, tot, 0.0)
        sp = 1
        while sp < nblk:
            w = w + pltpu.roll(w, sp, 0)
            sp *= 2
        off = off + jnp.where(bm > t, w, 0.0)
    return jnp.reshape(acc + off, (RB, ncols))


def _phase_c_body(cl_ref, ci_ref, cnt_ref, m_ref, z_ref, t_ref,
                  tk_ref, tp_ref, mp_ref, u_ref, o_ref):
    lane = lax.broadcasted_iota(jnp.int32, (RB, CAND), 1)
    cnt = cnt_ref[...][:, 0:1]
    t = t_ref[...][:, 0:1]
    m = m_ref[...][:, 0:1]
    valid = lane < cnt

    s = cl_ref[...] / t
    e = jnp.where(valid, jnp.exp(s - m), 0.0)
    z = z_ref[...][:, 0:1]
    p = jnp.where(valid, e / z, 0.0)
    ix = jnp.where(valid, ci_ref[...], jnp.int32(0x7FFFFFFF))

    for k in [2 << i for i in range(12)]:
        d = k // 2
        while d >= 1:
            pu = pltpu.roll(p, CAND - d, 1)
            pdn = pltpu.roll(p, d, 1)
            iu = pltpu.roll(ix, CAND - d, 1)
            idn = pltpu.roll(ix, d, 1)
            up_sel = (lane & d) == 0
            pb = jnp.where(up_sel, pu, pdn)
            ib = jnp.where(up_sel, iu, idn)
            dir_i = ((lane & k) == 0) == up_sel
            self_wins = jnp.logical_or(
                p > pb, jnp.logical_and(p == pb, ix < ib))
            keep = dir_i == self_wins
            p = jnp.where(keep, p, pb)
            ix = jnp.where(keep, ix, ib)
            d //= 2

    ph = p[:, :HALF]
    ixh = ix[:, :HALF]
    laneh = lane[:, :HALF]
    psum = _ref_cumsum(ph, HALF)
    pmax = p[:, 0:1]
    p1 = jnp.where((psum - ph) > tp_ref[...][:, 0:1], 0.0, ph)
    p2 = jnp.where(laneh >= tk_ref[...][:, 0:1], 0.0, p1)
    p3 = jnp.where(p2 < pmax * mp_ref[...][:, 0:1], 0.0, p2)
    mx = jnp.max(p3, axis=1, keepdims=True)
    pr = p3 / mx
    cdf = _ref_cumsum(pr, HALF)
    target = u_ref[...][:, 0:1] * cdf[:, HALF - 1:HALF]
    j = jnp.sum((cdf < target).astype(jnp.int32), axis=1, keepdims=True)
    j = jnp.clip(j, 0, V - 1)
    tok = jnp.sum(jnp.where(laneh == j, ixh, 0), axis=1, keepdims=True)
    o_ref[...] = jnp.broadcast_to(tok, (RB, 128))


def _phase_c(cl, ci, cnt, m128, z128, t128, tk128, tp128, mp128, u128):
    spec_cand = pl.BlockSpec((RB, CAND), lambda i: (i, 0))
    spec_128 = pl.BlockSpec((RB, 128), lambda i: (i, 0))
    spec_16 = pl.BlockSpec((RB, 16), lambda i: (i, 0))
    return pl.pallas_call(
        _phase_c_body,
        grid=(B // RB,),
        in_specs=[spec_cand, spec_cand, spec_16, spec_128, spec_128,
                  spec_128, spec_128, spec_128, spec_128, spec_128],
        out_specs=spec_128,
        out_shape=jax.ShapeDtypeStruct((B, 128), jnp.int32),
    )(cl, ci, cnt, m128, z128, t128, tk128, tp128, mp128, u128)


def kernel(logits, temperatures, top_ks, top_ps, min_ps, u):
    t128 = jnp.broadcast_to(temperatures.astype(jnp.float32), (B, 128))
    t8 = jnp.broadcast_to(jnp.reshape(temperatures.astype(jnp.float32),
                                      (1, B)), (8, B))
    m8, z8 = _phase_a(logits.T, t8)
    m128 = jnp.broadcast_to(m8[0][:, None], (B, 128))
    z128 = jnp.broadcast_to(z8[0][:, None], (B, 128))

    mt16 = (m128[:, :16] * t128[:, :16]).astype(jnp.float32)
    k16 = jnp.broadcast_to(
        (top_ks.astype(jnp.int32) + SLACK)[:, None], (B, 16))
    cl, ci, cnt = _sc_select(logits.reshape(-1), mt16, k16)
    cl = cl.reshape(B, CBUF)[:, :CAND]
    ci = ci.reshape(B, CBUF)[:, :CAND]

    tk128 = jnp.broadcast_to(top_ks.astype(jnp.int32)[:, None], (B, 128))
    tp128 = jnp.broadcast_to(top_ps.astype(jnp.float32)[:, None], (B, 128))
    mp128 = jnp.broadcast_to(min_ps.astype(jnp.float32)[:, None], (B, 128))
    u128 = jnp.broadcast_to(u.astype(jnp.float32)[:, None], (B, 128))

    out = _phase_c(cl, ci, cnt, m128, z128, t128, tk128, tp128, mp128, u128)
    return out[:, 0]


# speedup vs baseline: 8.7986x; 1.1101x over previous
import functools

import jax
import jax.numpy as jnp
from jax import lax
from jax.experimental import pallas as pl
from jax.experimental.pallas import tpu as pltpu
from jax.experimental.pallas import tpu_sc as plsc

B = 128
V = 100000
NB = 1024
CH = 10000
CBUF = 4352
CAND = 4096
SLACK = 64
ROWS_PER_W = 4
CHUNK_A = 7696
NCH_A = (V + CHUNK_A - 1) // CHUNK_A


def _phase_a_body(l_ref, t_ref, m_ref, z_ref, e_scr):
    ph = pl.program_id(0)
    ci = pl.program_id(1)
    row = lax.broadcasted_iota(jnp.int32, (CHUNK_A, B), 0) + ci * CHUNK_A
    valid = row < V
    t = t_ref[...][0:1, :]
    s = l_ref[...] / t

    @pl.when(jnp.logical_and(ph == 0, ci == 0))
    def _():
        m_ref[...] = jnp.full((8, B), -jnp.inf, jnp.float32)

    @pl.when(ph == 0)
    def _():
        sm = jnp.where(valid, s, -jnp.inf)
        cmax = jnp.max(sm, axis=0, keepdims=True)
        m_ref[...] = jnp.maximum(m_ref[...], jnp.broadcast_to(cmax, (8, B)))

    @pl.when(jnp.logical_and(ph == 1, ci == 0))
    def _():
        z_ref[...] = jnp.zeros((8, B), jnp.float32)

    @pl.when(ph == 1)
    def _():
        m = m_ref[...][0:1, :]
        e_scr[...] = jnp.where(valid, jnp.exp(s - m), 0.0)

        def body(j, acc):
            return acc + e_scr[pl.ds(j * 8, 8), :]
        acc = lax.fori_loop(0, CHUNK_A // 8, body,
                            jnp.zeros((8, B), jnp.float32))
        bb = acc + pltpu.roll(acc, 4, 0)
        cc = bb + pltpu.roll(bb, 2, 0)
        dd = cc + pltpu.roll(cc, 1, 0)
        z_ref[...] = z_ref[...] + jnp.broadcast_to(dd[0:1, :], (8, B))


def _phase_a(logits_t, t8):
    return pl.pallas_call(
        _phase_a_body,
        grid=(2, NCH_A),
        in_specs=[
            pl.BlockSpec((CHUNK_A, B), lambda p, i: (i, 0)),
            pl.BlockSpec((8, B), lambda p, i: (0, 0)),
        ],
        out_specs=[
            pl.BlockSpec((8, B), lambda p, i: (0, 0)),
            pl.BlockSpec((8, B), lambda p, i: (0, 0)),
        ],
        out_shape=[
            jax.ShapeDtypeStruct((8, B), jnp.float32),
            jax.ShapeDtypeStruct((8, B), jnp.float32),
        ],
        scratch_shapes=[pltpu.VMEM((CHUNK_A, B), jnp.float32)],
    )(logits_t, t8)


def _sc_select_body(lg, mt16, k16, cl, ci, cnt,
                    rowbuf, hist, clb, cib, msc, ksc, csc):
    wid = lax.axis_index("s") * 2 + lax.axis_index("c")
    scale = jnp.float32(NB / 72.0)

    def row_body(r, _carry):
        row = wid * ROWS_PER_W + r
        base = row * V
        pltpu.sync_copy(mt16.at[row], msc)
        pltpu.sync_copy(k16.at[row], ksc)
        mt_vec = msc[...]
        k_vec = ksc[...]
        pltpu.sync_copy(lg.at[pl.ds(base, V)], rowbuf)

        def bucket_of(lv):
            bk = ((mt_vec - lv) * scale).astype(jnp.int32)
            return jnp.clip(bk, 0, NB - 1)

        def zb(i, c):
            hist[pl.ds(i * 16, 16)] = jnp.zeros((16,), jnp.int32)
            return c
        lax.fori_loop(0, NB // 16, zb, 0)

        ones = jnp.ones((16,), jnp.int32)

        def vb_hist(i, cc):
            b0 = bucket_of(rowbuf[pl.ds(i * 32, 16)])
            b1 = bucket_of(rowbuf[pl.ds(i * 32 + 16, 16)])
            plsc.addupdate_scatter(hist, [b0], ones)
            plsc.addupdate_scatter(hist, [b1], ones)
            return cc
        lax.fori_loop(0, V // 32, vb_hist, 0)

        sentinel = jnp.full((16,), NB, jnp.int32)

        def scan_body(i, carry):
            csum, bstar = carry
            h = hist[pl.ds(i * 16, 16)]
            hc = plsc.cumsum(h) + csum
            cond = hc >= k_vec
            anyv = plsc.all_reduce_population_count(cond) > 0
            ffs = plsc.all_reduce_ffs(cond)
            cand_b = ffs + i * 16
            bstar = jnp.where(jnp.logical_and(bstar == NB, anyv), cand_b, bstar)
            return csum + jnp.sum(h), bstar
        _, bstar = lax.fori_loop(0, NB // 16, scan_body,
                                 (jnp.int32(0), sentinel))

        def vb_sel(i, off):
            lv0 = rowbuf[pl.ds(i * 32, 16)]
            lv1 = rowbuf[pl.ds(i * 32 + 16, 16)]
            m0 = bucket_of(lv0) <= bstar
            m1 = bucket_of(lv1) <= bstar
            iv = lax.iota(jnp.int32, 16)
            off0 = jnp.minimum(off, CBUF - 16)
            plsc.store_compressed(clb.at[pl.ds(off0, 16)], lv0, mask=m0)
            plsc.store_compressed(cib.at[pl.ds(off0, 16)], iv + i * 32,
                                  mask=m0)
            off1 = off + jnp.sum(m0.astype(jnp.int32))
            off1c = jnp.minimum(off1, CBUF - 16)
            plsc.store_compressed(clb.at[pl.ds(off1c, 16)], lv1, mask=m1)
            plsc.store_compressed(cib.at[pl.ds(off1c, 16)], iv + i * 32 + 16,
                                  mask=m1)
            return off1 + jnp.sum(m1.astype(jnp.int32))
        off = lax.fori_loop(0, V // 32, vb_sel, jnp.int32(0))

        pltpu.sync_copy(clb, cl.at[pl.ds(row * CBUF, CBUF)])
        pltpu.sync_copy(cib, ci.at[pl.ds(row * CBUF, CBUF)])
        csc[...] = jnp.zeros((16,), jnp.int32) + off
        pltpu.sync_copy(csc, cnt.at[row])
        return _carry

    lax.fori_loop(0, ROWS_PER_W, row_body, 0)


def _sc_select(lg_flat, mt16, k16):
    mesh = plsc.VectorSubcoreMesh(core_axis_name="c", subcore_axis_name="s")
    fn = functools.partial(
        pl.kernel,
        mesh=mesh,
        compiler_params=pltpu.CompilerParams(needs_layout_passes=False),
        out_type=[
            jax.ShapeDtypeStruct((B * CBUF,), jnp.float32),
            jax.ShapeDtypeStruct((B * CBUF,), jnp.int32),
            jax.ShapeDtypeStruct((B, 16), jnp.int32),
        ],
        scratch_types=[
            pltpu.VMEM((V,), jnp.float32),
            pltpu.VMEM((NB,), jnp.int32),
            pltpu.VMEM((CBUF,), jnp.float32),
            pltpu.VMEM((CBUF,), jnp.int32),
            pltpu.VMEM((16,), jnp.float32),
            pltpu.VMEM((16,), jnp.int32),
            pltpu.VMEM((16,), jnp.int32),
        ],
    )(_sc_select_body)
    return fn(lg_flat, mt16, k16)


RB = 32
HALF = CAND // 2
def _ref_cumsum(x, ncols):
    nblk = ncols // 128
    R = RB * nblk
    xb = jnp.reshape(x, (R, 128))
    lane = lax.broadcasted_iota(jnp.int32, (R, 128), 1)
    bm = lax.broadcasted_iota(jnp.int32, (R, 128), 0) & (nblk - 1)
    acc = jnp.broadcast_to(xb[:, 0:1], (R, 128))
    for t in range(1, 128):
        bv = jnp.broadcast_to(xb[:, t:t + 1], (R, 128))
        acc = acc + jnp.where(lane >= t, bv, 0.0)
    tot = jnp.broadcast_to(acc[:, 127:128], (R, 128))
    off = jnp.zeros((R, 128), jnp.float32)
    for t in range(nblk - 1):
        w = jnp.where(bm == t, tot, 0.0)
        sp = 1
        while sp < nblk:
            w = w + pltpu.roll(w, sp, 0)
            sp *= 2
        off = off + jnp.where(bm > t, w, 0.0)
    return jnp.reshape(acc + off, (RB, ncols))


def _phase_c_body(cl_ref, ci_ref, cnt_ref, m_ref, z_ref, t_ref,
                  tk_ref, tp_ref, mp_ref, u_ref, o_ref):
    lane = lax.broadcasted_iota(jnp.int32, (RB, CAND), 1)
    cnt = cnt_ref[...][:, 0:1]
    t = t_ref[...][:, 0:1]
    m = m_ref[...][:, 0:1]
    valid = lane < cnt

    s = cl_ref[...] / t
    e = jnp.where(valid, jnp.exp(s - m), 0.0)
    z = z_ref[...][:, 0:1]
    p = jnp.where(valid, e / z, 0.0)
    ix = jnp.where(valid, ci_ref[...], jnp.int32(0x7FFFFFFF))

    for k in [2 << i for i in range(12)]:
        d = k // 2
        while d >= 1:
            pu = pltpu.roll(p, CAND - d, 1)
            pdn = pltpu.roll(p, d, 1)
            iu = pltpu.roll(ix, CAND - d, 1)
            idn = pltpu.roll(ix, d, 1)
            up_sel = (lane & d) == 0
            pb = jnp.where(up_sel, pu, pdn)
            ib = jnp.where(up_sel, iu, idn)
            dir_i = ((lane & k) == 0) == up_sel
            self_wins = jnp.logical_or(
                p > pb, jnp.logical_and(p == pb, ix < ib))
            keep = dir_i == self_wins
            p = jnp.where(keep, p, pb)
            ix = jnp.where(keep, ix, ib)
            d //= 2

    ph = p[:, :HALF]
    ixh = ix[:, :HALF]
    laneh = lane[:, :HALF]
    psum = _ref_cumsum(ph, HALF)
    pmax = p[:, 0:1]
    p1 = jnp.where((psum - ph) > tp_ref[...][:, 0:1], 0.0, ph)
    p2 = jnp.where(laneh >= tk_ref[...][:, 0:1], 0.0, p1)
    p3 = jnp.where(p2 < pmax * mp_ref[...][:, 0:1], 0.0, p2)
    mx = jnp.max(p3, axis=1, keepdims=True)
    pr = p3 / mx
    cdf = _ref_cumsum(pr, HALF)
    target = u_ref[...][:, 0:1] * cdf[:, HALF - 1:HALF]
    j = jnp.sum((cdf < target).astype(jnp.int32), axis=1, keepdims=True)
    j = jnp.clip(j, 0, V - 1)
    tok = jnp.sum(jnp.where(laneh == j, ixh, 0), axis=1, keepdims=True)
    o_ref[...] = jnp.broadcast_to(tok, (RB, 128))


def _phase_c(cl, ci, cnt, m128, z128, t128, tk128, tp128, mp128, u128):
    spec_cand = pl.BlockSpec((RB, CAND), lambda i: (i, 0))
    spec_128 = pl.BlockSpec((RB, 128), lambda i: (i, 0))
    spec_16 = pl.BlockSpec((RB, 16), lambda i: (i, 0))
    return pl.pallas_call(
        _phase_c_body,
        grid=(B // RB,),
        in_specs=[spec_cand, spec_cand, spec_16, spec_128, spec_128,
                  spec_128, spec_128, spec_128, spec_128, spec_128],
        out_specs=spec_128,
        out_shape=jax.ShapeDtypeStruct((B, 128), jnp.int32),
    )(cl, ci, cnt, m128, z128, t128, tk128, tp128, mp128, u128)


def kernel(logits, temperatures, top_ks, top_ps, min_ps, u):
    t128 = jnp.broadcast_to(temperatures.astype(jnp.float32), (B, 128))
    t8 = jnp.broadcast_to(jnp.reshape(temperatures.astype(jnp.float32),
                                      (1, B)), (8, B))
    m8, z8 = _phase_a(logits.T, t8)
    m128 = jnp.broadcast_to(m8[0][:, None], (B, 128))
    z128 = jnp.broadcast_to(z8[0][:, None], (B, 128))

    mt16 = (m128[:, :16] * t128[:, :16]).astype(jnp.float32)
    k16 = jnp.broadcast_to(
        (top_ks.astype(jnp.int32) + SLACK)[:, None], (B, 16))
    cl, ci, cnt = _sc_select(logits.reshape(-1), mt16, k16)
    cl = cl.reshape(B, CBUF)[:, :CAND]
    ci = ci.reshape(B, CBUF)[:, :CAND]

    tk128 = jnp.broadcast_to(top_ks.astype(jnp.int32)[:, None], (B, 128))
    tp128 = jnp.broadcast_to(top_ps.astype(jnp.float32)[:, None], (B, 128))
    mp128 = jnp.broadcast_to(min_ps.astype(jnp.float32)[:, None], (B, 128))
    u128 = jnp.broadcast_to(u.astype(jnp.float32)[:, None], (B, 128))

    out = _phase_c(cl, ci, cnt, m128, z128, t128, tk128, tp128, mp128, u128)
    return out[:, 0]
